# Initial kernel scaffold; baseline (speedup 1.0000x reference)
import jax
import jax.numpy as jnp
from jax.experimental import pallas as pl


def kernel(node_s, node_v, edge_s, edge_v, dist, vctr_norm, edge_index, W_nn, b_nn, W_ep, b_ep, gm_W1, gm_b1, gm_Wo, gm_bo, gm_Wg, gm_bg, W_ev, b_ev, W_nvout, W_nvch, b_nvch, W_nvproj, W_nsproj, b_nsproj, ln_g, ln_b, cn_scale):
    raise NotImplementedError("write your pallas kernel here")


# speedup vs baseline: 6.6247x; 6.6247x over previous
import functools

import jax
import jax.numpy as jnp
from jax import lax
from jax.experimental import pallas as pl
from jax.experimental.pallas import tpu as pltpu
from jax.experimental.pallas import tpu_sc as plsc

F = 128
CUTOFF = 5.0


def _silu(x):
    return x * jax.nn.sigmoid(x)


def _cutoff(d):
    return 0.5 * (jnp.cos(jnp.pi * d / CUTOFF) + 1.0) * (d < CUTOFF).astype(jnp.float32)


def _edge_kernel_body(ns_src, ns_dst, nv_src, es, ev, dist, vctr,
                      W_nn, b_nn, W_ep, b_ep, gm_W1, gm_b1, gm_Wo, gm_bo,
                      gm_Wg, gm_bg, W_ev, b_ev,
                      es_aug_o, ev_upd_o, es_out_o, ev_out_o):
    x_es = es[...]
    x_ev = ev[...]
    x_nv = nv_src[...]
    w_nn = W_nn[...]
    nn = (jnp.dot(ns_src[...], w_nn[:F, :], preferred_element_type=jnp.float32)
          + jnp.dot(ns_dst[...], w_nn[F:, :], preferred_element_type=jnp.float32)
          + b_nn[...])
    em = nn * (jnp.dot(x_es, W_ep[...], preferred_element_type=jnp.float32) + b_ep[...])
    h = _silu(jnp.dot(em, gm_W1[...], preferred_element_type=jnp.float32) + gm_b1[...])
    es_upd = ((jnp.dot(h, gm_Wo[...], preferred_element_type=jnp.float32) + gm_bo[...])
              * jax.nn.sigmoid(jnp.dot(h, gm_Wg[...], preferred_element_type=jnp.float32)
                               + gm_bg[...]))
    cut = _cutoff(dist[...])
    es_upd = es_upd * cut
    vc = jnp.dot(es_upd, W_ev[...], preferred_element_type=jnp.float32) + b_ev[...]
    node_ch = vc[:, :F]
    edge_ch = vc[:, F:2 * F]
    rel_ch = vc[:, 2 * F:]
    B = x_es.shape[0]
    x_vc = vctr[...]
    evs = []
    for c in range(3):
        ev_c = (x_nv[:, c * F:(c + 1) * F] * node_ch
                + x_ev[:, c * F:(c + 1) * F] * edge_ch
                + x_vc[:, c:c + 1] * rel_ch) * cut
        evs.append(ev_c)
    ev_upd = jnp.concatenate(evs, axis=1)
    ones_col = jnp.ones((B, 1), jnp.float32)
    zeros_pad = jnp.zeros((B, 127), jnp.float32)
    es_aug_o[...] = jnp.concatenate([es_upd, ones_col, zeros_pad], axis=1)
    ev_upd_o[...] = ev_upd
    es_out_o[...] = es_upd + x_es
    ev_out_o[...] = ev_upd + x_ev


def _edge_tc(ns_src, ns_dst, nv_src, es2, ev2, dist, vctr, params,
             interpret=False):
    E = es2.shape[0]
    B = 640
    grid = E // B
    (W_nn, b_nn, W_ep, b_ep, gm_W1, gm_b1, gm_Wo, gm_bo, gm_Wg, gm_bg,
     W_ev, b_ev) = params
    row = lambda n: pl.BlockSpec((B, n), lambda i: (i, 0))
    full = lambda a: pl.BlockSpec(a.shape, lambda i: (0,) * a.ndim)
    in_specs = [row(F), row(F), row(3 * F), row(F), row(3 * F), row(1), row(3)] + \
               [full(w) for w in params]
    out_specs = [row(F + 128), row(3 * F), row(F), row(3 * F)]
    out_shape = [
        jax.ShapeDtypeStruct((E, F + 128), jnp.float32),
        jax.ShapeDtypeStruct((E, 3 * F), jnp.float32),
        jax.ShapeDtypeStruct((E, F), jnp.float32),
        jax.ShapeDtypeStruct((E, 3 * F), jnp.float32),
    ]
    return pl.pallas_call(
        _edge_kernel_body,
        grid=(grid,),
        in_specs=in_specs,
        out_specs=out_specs,
        out_shape=out_shape,
        interpret=interpret,
    )(ns_src, ns_dst, nv_src, es2, ev2, dist, vctr, *params)


def _node_kernel_body(ev_p0, ev_p1, es_p0, es_p1, ns, nv,
                      W_nvout, W_nvch, b_nvch, W_nvproj, W_nsproj, b_nsproj,
                      ln_g, ln_b, cn_scale,
                      ns_out_o, nv_out_o):
    x_es_aug = es_p0[...] + es_p1[...]
    es_sum = x_es_aug[:, :F]
    deg = x_es_aug[:, F:F + 1]
    denom = jnp.maximum(deg, 1.0)
    inv = 1.0 / denom
    n_es = es_sum * inv
    x_ev = ev_p0[...] + ev_p1[...]
    w_out = W_nvout[...]
    nvo = [jnp.dot(x_ev[:, c * F:(c + 1) * F] * inv, w_out,
                   preferred_element_type=jnp.float32) for c in range(3)]
    o3sq = sum(o[:, 2 * F:] ** 2 for o in nvo)
    o3_norm = jnp.sqrt(o3sq)
    w_ch = W_nvch[...]
    v_channel = (jnp.dot(n_es, w_ch[:F, :], preferred_element_type=jnp.float32)
                 + jnp.dot(o3_norm, w_ch[F:, :], preferred_element_type=jnp.float32)
                 + b_nvch[...])
    nvu = [o[:, :F] * v_channel + o[:, F:2 * F] for o in nvo]
    w_proj = W_nvproj[...]
    nvp = [jnp.dot(u, w_proj, preferred_element_type=jnp.float32) for u in nvu]
    ns_proj = _silu(jnp.dot(n_es, W_nsproj[...], preferred_element_type=jnp.float32)
                    + b_nsproj[...])
    nv_dot = sum(p[:, :F] * p[:, F:] for p in nvp)
    n_s_update = nv_dot * ns_proj[:, :F] + ns_proj[:, F:]
    ns_res = n_s_update + ns[...]
    mu = jnp.mean(ns_res, axis=1, keepdims=True)
    var = jnp.mean((ns_res - mu) ** 2, axis=1, keepdims=True)
    ns_out_o[...] = (ns_res - mu) / jnp.sqrt(var + 1e-5) * ln_g[...] + ln_b[...]
    x_nv = nv[...]
    nv_res = [nvu[c] + x_nv[:, c * F:(c + 1) * F] for c in range(3)]
    vnorm = jnp.sqrt(sum(r ** 2 for r in nv_res))
    scale = cn_scale[...] / (vnorm + 1e-8)
    nv_out_o[...] = jnp.concatenate([r * scale for r in nv_res], axis=1)


def _node_tc(ev_part, es_part, ns2, nv2, params, interpret=False):
    N = ns2.shape[0]
    B = 400
    grid = N // B
    off = _RPAD // B
    row = lambda n: pl.BlockSpec((B, n), lambda i: (i, 0))
    row_off = lambda n: pl.BlockSpec((B, n), lambda i: (i + off, 0))
    full = lambda a: pl.BlockSpec(a.shape, lambda i: (0,) * a.ndim)
    in_specs = [row(3 * F), row_off(3 * F), row(F + 128), row_off(F + 128),
                row(F), row(3 * F)] + [full(w) for w in params]
    out_specs = [row(F), row(3 * F)]
    out_shape = [
        jax.ShapeDtypeStruct((N, F), jnp.float32),
        jax.ShapeDtypeStruct((N, 3 * F), jnp.float32),
    ]
    return pl.pallas_call(
        _node_kernel_body,
        grid=(grid,),
        in_specs=in_specs,
        out_specs=out_specs,
        out_shape=out_shape,
        interpret=interpret,
    )(ev_part, ev_part, es_part, es_part, ns2, nv2, *params)


_NC = 2
_NS = 16
_NW = _NC * _NS


def _sc_mesh():
    return plsc.VectorSubcoreMesh(core_axis_name="c", subcore_axis_name="s")


def _sc_gather(ns2, nv2, src, dst):
    E = src.shape[0]
    EW = E // _NW
    C = 128
    nfull = EW // C
    rem = EW - nfull * C

    @functools.partial(
        pl.kernel,
        mesh=_sc_mesh(),
        out_type=[
            jax.ShapeDtypeStruct((E, F), jnp.float32),
            jax.ShapeDtypeStruct((E, F), jnp.float32),
            jax.ShapeDtypeStruct((E, 3 * F), jnp.float32),
        ],
        scratch_types=[
            pltpu.VMEM((C,), jnp.int32),
            pltpu.VMEM((C, F), jnp.float32),
            pltpu.VMEM((C, 3 * F), jnp.float32),
            pltpu.SemaphoreType.DMA,
        ],
    )
    def gather_k(ns_hbm, nv_hbm, src_hbm, dst_hbm, o_ns_src, o_ns_dst,
                 o_nv_src, idx_v, r1, r3, sem):
        wid = lax.axis_index("s") * _NC + lax.axis_index("c")
        base = wid * EW

        def do_chunk(st):
            pltpu.sync_copy(src_hbm.at[pl.ds(st, C)], idx_v)
            pltpu.async_copy(ns_hbm.at[idx_v], r1, sem).wait()
            pltpu.sync_copy(r1, o_ns_src.at[pl.ds(st, C)])
            pltpu.async_copy(nv_hbm.at[idx_v], r3, sem).wait()
            pltpu.sync_copy(r3, o_nv_src.at[pl.ds(st, C)])
            pltpu.sync_copy(dst_hbm.at[pl.ds(st, C)], idx_v)
            pltpu.async_copy(ns_hbm.at[idx_v], r1, sem).wait()
            pltpu.sync_copy(r1, o_ns_dst.at[pl.ds(st, C)])

        def chunk(i, _):
            do_chunk(base + i * C)
            return 0

        lax.fori_loop(0, nfull, chunk, 0)
        if rem:
            do_chunk(base + EW - C)

    return gather_k(ns2, nv2, src, dst)


_RPAD = 10400


def _sc_scatter(rows, dst, N, D):
    E = rows.shape[0]
    RPAD = _RPAD
    EW = E // _NW
    C = 128
    nfull = EW // C
    rem = EW - nfull * C

    @functools.partial(
        pl.kernel,
        mesh=_sc_mesh(),
        out_type=[jax.ShapeDtypeStruct((2 * RPAD, D), jnp.float32)],
        scratch_types=[
            pltpu.VMEM((C, D), jnp.float32),
            pltpu.VMEM((C,), jnp.int32),
            pltpu.VMEM((1, C), jnp.int32),
            pltpu.SemaphoreType.DMA,
        ],
    )
    def scatter_k(rows_hbm, dst_hbm, out_hbm, rv, dsts, lidx, sem):
        core = lax.axis_index("c")
        s = lax.axis_index("s")
        wid = s * _NC + core
        obase = core * RPAD

        def zrow(i, _):
            for kk in range(D // 16):
                rv[i, pl.ds(kk * 16, 16)] = jnp.zeros((16,), jnp.float32)
            return 0

        lax.fori_loop(0, C, zrow, 0)

        def zchunk(i, _):
            pltpu.sync_copy(rv, out_hbm.at[pl.ds(obase + s * 640 + i * C, C)])
            return 0

        lax.fori_loop(0, 5, zchunk, 0)

        @pl.when(s == _NS - 1)
        def _():
            def zchunk2(i, _):
                pltpu.sync_copy(
                    rv, out_hbm.at[pl.ds(obase + 9600 + 640 + i * C, C)])
                return 0

            lax.fori_loop(0, 1, zchunk2, 0)
            pltpu.sync_copy(rv.at[pl.ds(0, 32)],
                            out_hbm.at[pl.ds(obase + RPAD - 32, 32)])

        plsc.subcore_barrier()

        def localize(skip):
            for kk in range(C // 16):
                v = dsts[pl.ds(kk * 16, 16)]
                ok = v >= 0
                if kk * 16 + 16 <= skip:
                    ok = ok & (v < 0)
                lidx[0, pl.ds(kk * 16, 16)] = obase + jnp.where(
                    ok, v, N + (v & 63))

        def do_chunk(eb, skip):
            pltpu.sync_copy(dst_hbm.at[pl.ds(eb, C)], dsts)
            pltpu.sync_copy(rows_hbm.at[pl.ds(eb, C)], rv)
            localize(skip)
            pltpu.async_copy(rv, out_hbm.at[lidx.at[0]], sem, add=True).wait()

        def chunk(i, _):
            do_chunk(wid * EW + i * C, 0)
            return 0

        lax.fori_loop(0, nfull, chunk, 0)
        if rem:
            do_chunk(wid * EW + EW - C, C - rem)

    return scatter_k(rows, dst)[0]


def _gather_jnp(node_s2, node_v2, src, dst):
    return node_s2[src], node_s2[dst], node_v2[src]


def _scatter_jnp(es_aug, ev_upd, dst, N):
    sum_es_aug = jax.ops.segment_sum(es_aug, dst, num_segments=N)
    sum_ev = jax.ops.segment_sum(ev_upd, dst, num_segments=N)
    pad = lambda x: jnp.concatenate(
        [x, jnp.zeros((2 * _RPAD - N, x.shape[1]), jnp.float32)], axis=0)
    return pad(sum_ev), pad(sum_es_aug)


def kernel(node_s, node_v, edge_s, edge_v, dist, vctr_norm, edge_index,
           W_nn, b_nn, W_ep, b_ep, gm_W1, gm_b1, gm_Wo, gm_bo, gm_Wg, gm_bg,
           W_ev, b_ev, W_nvout, W_nvch, b_nvch, W_nvproj, W_nsproj, b_nsproj,
           ln_g, ln_b, cn_scale):
    N = node_s.shape[0]
    E = edge_s.shape[0]
    src = edge_index[0]
    dst = edge_index[1]
    ns2 = node_s.reshape(N, F)
    nv2 = node_v.reshape(N, 3 * F)
    es2 = edge_s.reshape(E, F)
    ev2 = edge_v.reshape(E, 3 * F)

    ns_src, ns_dst, nv_src = _sc_gather(ns2, nv2, src, dst)

    edge_params = (W_nn, b_nn.reshape(1, F), W_ep, b_ep.reshape(1, F),
                   gm_W1, gm_b1.reshape(1, F), gm_Wo, gm_bo.reshape(1, F),
                   gm_Wg, gm_bg.reshape(1, F), W_ev, b_ev.reshape(1, 3 * F))
    es_aug, ev_upd, edge_s_out, edge_v_out = _edge_tc(
        ns_src, ns_dst, nv_src, es2, ev2, dist, vctr_norm, edge_params)

    ev_part, es_part = _scatter_jnp(es_aug, ev_upd, dst, N)

    node_params = (W_nvout, W_nvch, b_nvch.reshape(1, F), W_nvproj,
                   W_nsproj, b_nsproj.reshape(1, 2 * F),
                   ln_g.reshape(1, F), ln_b.reshape(1, F),
                   cn_scale.reshape(1, F))
    ns_out, nv_out = _node_tc(ev_part, es_part, ns2, nv2, node_params)

    return (ns_out.reshape(N, 1, F), nv_out.reshape(N, 3, F),
            edge_s_out.reshape(E, 1, F), edge_v_out.reshape(E, 3, F))
